# baseline (speedup 1.0000x reference)
import functools

import jax
import jax.numpy as jnp
from jax import lax
from jax.experimental import pallas as pl
from jax.experimental.pallas import tpu as pltpu
from jax.experimental.pallas import tpu_sc as plsc

_WIN = 128
_NW = 32
_NSUB = 16


def _mesh():
    return plsc.VectorSubcoreMesh(core_axis_name="c", subcore_axis_name="s")


def _deg_call(col2, w2, n_pad):
    ep = w2.shape[0] * w2.shape[1]
    per_w = ep // _NW
    nwin = per_w // _WIN

    def body(col_h, w_h, out_h, acc, col_v, w_v, zero_v, sem):
        c = lax.axis_index("c")
        s = lax.axis_index("s")
        wid = c * _NSUB + s

        def _zero(j, _):
            zero_v[pl.ds(j * 16, 16)] = jnp.zeros((16,), jnp.float32)
            return 0

        lax.fori_loop(0, n_pad // 16, _zero, 0)

        @pl.when(s == 0)
        def _():
            pltpu.sync_copy(zero_v, acc)

        plsc.subcore_barrier()

        wbase = wid * nwin
        pltpu.sync_copy(col_h.at[pl.ds(wbase, nwin)], col_v)
        pltpu.sync_copy(w_h.at[pl.ds(wbase, nwin)], w_v)

        def win(k, _):
            pltpu.async_copy(
                w_v.at[k], acc.at[col_v.at[k]], sem, add=True)
            return 0

        lax.fori_loop(0, nwin, win, 0)

        def drain(k, _):
            pltpu.make_async_copy(
                w_v.at[0], acc.at[col_v.at[0]], sem).wait()
            return 0

        lax.fori_loop(0, nwin, drain, 0)
        plsc.subcore_barrier()

        @pl.when(s == 0)
        def _():
            pltpu.sync_copy(acc, out_h.at[c])

    return pl.kernel(
        body,
        out_type=jax.ShapeDtypeStruct((2, n_pad), jnp.float32),
        mesh=_mesh(),
        scratch_types=[
            pltpu.VMEM_SHARED((n_pad,), jnp.float32),
            pltpu.VMEM((nwin, _WIN), jnp.int32),
            pltpu.VMEM((nwin, _WIN), jnp.float32),
            pltpu.VMEM((n_pad,), jnp.float32),
            pltpu.SemaphoreType.DMA,
        ],
    )(col2, w2)


def _norm_call(row, col, w, dis):
    ep = row.shape[0]
    per_w = ep // _NW
    nwin = per_w // _WIN

    def body(row_h, col_h, w_h, dis_h, out_h, row_v, col_v, w_v,
             dr0, dr1, dc0, dc1, nv0, nv1, sg0, sg1, ss0, ss1):
        c = lax.axis_index("c")
        s = lax.axis_index("s")
        wid = c * _NSUB + s
        ebase = wid * per_w
        dr = (dr0, dr1)
        dc = (dc0, dc1)
        nv = (nv0, nv1)
        sg = (sg0, sg1)
        ss = (ss0, ss1)

        pltpu.sync_copy(row_h.at[pl.ds(ebase, per_w)], row_v)
        pltpu.sync_copy(col_h.at[pl.ds(ebase, per_w)], col_v)
        pltpu.sync_copy(w_h.at[pl.ds(ebase, per_w)], w_v)

        def issue_g(wl, b):
            pltpu.async_copy(
                dis_h.at[row_v.at[pl.ds(wl * _WIN, _WIN)]], dr[b], sg[b])
            pltpu.async_copy(
                dis_h.at[col_v.at[pl.ds(wl * _WIN, _WIN)]], dc[b], sg[b])

        def wait_g(b):
            pltpu.make_async_copy(
                dis_h.at[row_v.at[pl.ds(0, _WIN)]], dr[b], sg[b]).wait()
            pltpu.make_async_copy(
                dis_h.at[col_v.at[pl.ds(0, _WIN)]], dc[b], sg[b]).wait()

        def issue_s(wl, b):
            pltpu.async_copy(
                nv[b], out_h.at[pl.ds(ebase + wl * _WIN, _WIN)], ss[b])

        def wait_s(b):
            pltpu.make_async_copy(
                nv[b], out_h.at[pl.ds(ebase, _WIN)], ss[b]).wait()

        issue_g(0, 0)

        def pair(k0, _):
            for b in range(2):
                wl = 2 * k0 + b
                b2 = 1 - b

                @pl.when(wl >= 1)
                def _():
                    wait_s(b2)

                @pl.when(wl <= nwin - 2)
                def _():
                    issue_g(wl + 1, b2)

                wait_g(b)
                for q in range(_WIN // 16):
                    sl = pl.ds(q * 16, 16)
                    wsl = pl.ds(wl * _WIN + q * 16, 16)
                    nv[b][sl] = dr[b][sl] * w_v[wsl] * dc[b][sl]
                issue_s(wl, b)
            return 0

        lax.fori_loop(0, nwin // 2, pair, 0)
        wait_s(1)

    return pl.kernel(
        body,
        out_type=jax.ShapeDtypeStruct((ep,), jnp.float32),
        mesh=_mesh(),
        scratch_types=[
            pltpu.VMEM((per_w,), jnp.int32),
            pltpu.VMEM((per_w,), jnp.int32),
            pltpu.VMEM((per_w,), jnp.float32),
            pltpu.VMEM((_WIN,), jnp.float32),
            pltpu.VMEM((_WIN,), jnp.float32),
            pltpu.VMEM((_WIN,), jnp.float32),
            pltpu.VMEM((_WIN,), jnp.float32),
            pltpu.VMEM((_WIN,), jnp.float32),
            pltpu.VMEM((_WIN,), jnp.float32),
            pltpu.SemaphoreType.DMA,
            pltpu.SemaphoreType.DMA,
            pltpu.SemaphoreType.DMA,
            pltpu.SemaphoreType.DMA,
        ],
    )(row, col, w, dis)


def _agg_call(table, row, col2, norm, n_pad):
    n_nodes, ch = table.shape
    ep = row.shape[0]
    per_w = ep // _NW
    nwin = per_w // _WIN
    nseg = 4
    half = nwin // nseg
    hlen = half * _WIN
    rows_per_sub = n_pad // _NSUB

    zfull = rows_per_sub // _WIN
    zrem = rows_per_sub % _WIN

    def body(table_h, row_h, col_h, norm_h, out_h, part, row_v, col_v,
             norm_v, rows_v0, rows_v1, sg0, sg1, ss0, ss1):
        c = lax.axis_index("c")
        s = lax.axis_index("s")
        wid = c * _NSUB + s
        rows_b = (rows_v0, rows_v1)
        sg = (sg0, sg1)
        ss = (ss0, ss1)

        def _zero(j, _):
            for q in range(ch // 16):
                rows_v0[j, pl.ds(q * 16, 16)] = jnp.zeros((16,), jnp.float32)
            return 0

        lax.fori_loop(0, _WIN, _zero, 0)
        for k in range(zfull):
            pltpu.sync_copy(
                rows_v0, part.at[pl.ds(s * rows_per_sub + k * _WIN, _WIN)])
        if zrem:
            pltpu.sync_copy(
                rows_v0.at[pl.ds(0, zrem)],
                part.at[pl.ds(s * rows_per_sub + zfull * _WIN, zrem)])
        plsc.subcore_barrier()

        def issue_g(wl, b):
            return pltpu.async_copy(
                table_h.at[row_v.at[pl.ds(wl * _WIN, _WIN)]], rows_b[b], sg[b])

        def wait_g(b):
            pltpu.make_async_copy(
                table_h.at[row_v.at[pl.ds(0, _WIN)]], rows_b[b], sg[b]).wait()

        hw = _WIN // 2

        def issue_s(wl, b, piece):
            return pltpu.async_copy(
                rows_b[b].at[pl.ds(piece * hw, hw)],
                part.at[col_v.at[2 * wl + piece]], ss[b], add=True)

        def wait_s(b):
            for piece in range(2):
                pltpu.make_async_copy(
                    rows_b[b].at[pl.ds(piece * hw, hw)],
                    part.at[col_v.at[piece]], ss[b]).wait()

        def do_mul(wl, b):
            rv = rows_b[b]

            def mul16(kb, _):
                nv16 = norm_v[pl.ds(wl * _WIN + kb * 16, 16)]
                for j in range(16):
                    nv = jnp.full((16,), nv16[j], jnp.float32)
                    e = kb * 16 + j
                    for q in range(ch // 16):
                        sl = pl.ds(q * 16, 16)
                        rv[e, sl] = rv[e, sl] * nv
                return 0

            lax.fori_loop(0, hw // 16, mul16, 0)
            issue_s(wl, b, 0)
            lax.fori_loop(hw // 16, _WIN // 16, mul16, 0)
            issue_s(wl, b, 1)

        for h in range(nseg):
            ebase = wid * per_w + h * hlen
            wbase = wid * nwin + h * half
            pltpu.sync_copy(row_h.at[pl.ds(ebase, hlen)], row_v)
            pltpu.sync_copy(col_h.at[pl.ds(2 * wbase, 2 * half)], col_v)
            pltpu.sync_copy(norm_h.at[pl.ds(ebase, hlen)], norm_v)
            issue_g(0, 0)

            def pair(k0, _):
                for b in range(2):
                    wl = 2 * k0 + b
                    b2 = 1 - b

                    @pl.when(wl >= 1)
                    def _():
                        wait_s(b2)

                    @pl.when(wl <= half - 2)
                    def _():
                        issue_g(wl + 1, b2)

                    wait_g(b)
                    do_mul(wl, b)
                return 0

            lax.fori_loop(0, half // 2, pair, 0)
            wait_s(1)

        plsc.subcore_barrier()
        pltpu.sync_copy(
            part.at[pl.ds(s * rows_per_sub, rows_per_sub)],
            out_h.at[c].at[pl.ds(s * rows_per_sub, rows_per_sub)])

    return pl.kernel(
        body,
        out_type=jax.ShapeDtypeStruct((2, n_pad, ch), jnp.float32),
        mesh=_mesh(),
        scratch_types=[
            pltpu.VMEM_SHARED((n_pad, ch), jnp.float32),
            pltpu.VMEM((hlen,), jnp.int32),
            pltpu.VMEM((2 * half, _WIN // 2), jnp.int32),
            pltpu.VMEM((hlen,), jnp.float32),
            pltpu.VMEM((_WIN, ch), jnp.float32),
            pltpu.VMEM((_WIN, ch), jnp.float32),
            pltpu.SemaphoreType.DMA,
            pltpu.SemaphoreType.DMA,
            pltpu.SemaphoreType.DMA,
            pltpu.SemaphoreType.DMA,
        ],
    )(table, row, col2, norm)


_GRID = 10


def _lstm_tail(cc, c_prev, hid):
    i = jax.nn.sigmoid(cc[:, :hid])
    f = jax.nn.sigmoid(cc[:, hid:2 * hid])
    o = jax.nn.sigmoid(cc[:, 2 * hid:3 * hid])
    g = jnp.tanh(cc[:, 3 * hid:])
    c2 = f * c_prev + i * g
    return o * jnp.tanh(c2), c2


def _enc_step_body(px0, px1, xt, ph0, ph1, hp, cp, sw, w, b, h_out, c_out):
    hid = h_out.shape[1]
    cin = xt.shape[1]
    wf = w[...]
    aggx = px0[0] + px1[0] + sw[...] * xt[...]
    aggh = ph0[0] + ph1[0] + sw[...] * hp[...]
    cc = (jnp.dot(aggx, wf[:cin], preferred_element_type=jnp.float32)
          + jnp.dot(aggh, wf[cin:], preferred_element_type=jnp.float32)
          + b[...])
    h2, c2 = _lstm_tail(cc, cp[...], hid)
    h_out[...] = h2
    c_out[...] = c2


def _enc_step0_body(px0, px1, xt, sw, w, b, h_out, c_out):
    hid = h_out.shape[1]
    cin = xt.shape[1]
    wf = w[...]
    aggx = px0[0] + px1[0] + sw[...] * xt[...]
    cc = jnp.dot(aggx, wf[:cin], preferred_element_type=jnp.float32) + b[...]
    i = jax.nn.sigmoid(cc[:, :hid])
    o = jax.nn.sigmoid(cc[:, 2 * hid:3 * hid])
    g = jnp.tanh(cc[:, 3 * hid:])
    c2 = i * g
    h_out[...] = o * jnp.tanh(c2)
    c_out[...] = c2


def _dec_step_body(vx, ph0, ph1, hp, cp, sw, w, b, h_out, c_out):
    hid = h_out.shape[1]
    wf = w[...]
    aggh = ph0[0] + ph1[0] + sw[...] * hp[...]
    cc = (jnp.dot(vx[...], wf[:hid], preferred_element_type=jnp.float32)
          + jnp.dot(aggh, wf[hid:], preferred_element_type=jnp.float32)
          + b[...])
    h2, c2 = _lstm_tail(cc, cp[...], hid)
    h_out[...] = h2
    c_out[...] = c2


def _init_body(ph0, ph1, hp, pc0, pc1, cp, sw, w, b, h_out, c_out, v_out):
    hid = h_out.shape[1]
    wf = w[...]
    aggh = ph0[0] + ph1[0] + sw[...] * hp[...]
    aggc = pc0[0] + pc1[0] + sw[...] * cp[...]
    st = (jnp.dot(aggh, wf[:hid], preferred_element_type=jnp.float32)
          + jnp.dot(aggc, wf[hid:], preferred_element_type=jnp.float32)
          + b[...])
    st = jnp.where(st > 0, st, jnp.exp(jnp.minimum(st, 0.0)) - 1.0)
    h_out[...] = st[:, :hid]
    c_out[...] = st[:, hid:]
    v_out[...] = aggh


def _row_spec(n, ch):
    return pl.BlockSpec((n // _GRID, ch), lambda i: (i, 0))


def _full_spec(r, c):
    return pl.BlockSpec((r, c), lambda i: (0, 0))


def _tc_call(body, n, hid, in_specs, num_outs):
    return pl.pallas_call(
        body,
        grid=(_GRID,),
        in_specs=in_specs,
        out_specs=[_row_spec(n, hid)] * num_outs,
        out_shape=[jax.ShapeDtypeStruct((n, hid), jnp.float32)] * num_outs,
    )


def _dis_body(deg, dis_out, sw_out):
    d = deg[0:1, :] + deg[1:2, :] + 1.0
    dis_out[...] = lax.rsqrt(d)
    sw_out[...] = 1.0 / d


def kernel(x, edge_index, edge_attr, seq_len, W_enc, b_enc, W_init, b_init,
           W_dec, b_dec):
    t_enc, n, cin = x.shape
    hid = W_enc.shape[1] // 4
    out_c = W_dec.shape[1] // 4
    e = edge_index.shape[1]

    chunk = _NW * _WIN * 2
    ep = ((e + chunk - 1) // chunk) * chunk
    pad = ep - e
    row = edge_index[0]
    col = edge_index[1]
    w = edge_attr
    if pad:
        fill = (jnp.arange(pad, dtype=jnp.int32) * 97) % n
        row = jnp.concatenate([row, fill])
        col = jnp.concatenate([col, fill])
        w = jnp.concatenate([w, jnp.zeros((pad,), jnp.float32)])

    n_pad = ((n + 127) // 128) * 128

    col2 = col.reshape(-1, _WIN)
    degp = _deg_call(col2, w.reshape(-1, _WIN), n_pad)
    dis2, sw2 = pl.pallas_call(
        _dis_body,
        in_specs=[pl.BlockSpec((2, n_pad), lambda: (0, 0))],
        out_specs=[pl.BlockSpec((1, n_pad), lambda: (0, 0))] * 2,
        out_shape=[jax.ShapeDtypeStruct((1, n_pad), jnp.float32)] * 2,
    )(degp)
    dis = dis2.reshape(n_pad)
    sw = sw2.reshape(n_pad, 1)
    norm = _norm_call(row, col, w, dis)
    col4 = col.reshape(-1, _WIN // 2)

    agg = lambda tbl: _agg_call(tbl, row, col4, norm, n_pad)

    b_enc2 = b_enc.reshape(1, -1)
    b_init2 = b_init.reshape(1, -1)
    b_dec2 = b_dec.reshape(1, -1)

    r = n // _GRID
    rs = _row_spec(n, hid)
    ss = pl.BlockSpec((r, 1), lambda i: (i, 0))
    p0 = pl.BlockSpec((1, r, hid), lambda i: (0, i, 0))
    p1 = pl.BlockSpec((1, r, hid), lambda i: (1, i, 0))
    wspec_enc = _full_spec(cin + hid, 4 * hid)
    bspec_enc = _full_spec(1, 4 * hid)

    enc0 = _tc_call(_enc_step0_body, n, hid,
                    [p0, p1, rs, ss, wspec_enc, bspec_enc], 2)
    encs = _tc_call(_enc_step_body, n, hid,
                    [p0, p1, rs, p0, p1, rs, rs, ss, wspec_enc, bspec_enc], 2)
    init = pl.pallas_call(
        _init_body,
        grid=(_GRID,),
        in_specs=[p0, p1, rs, p0, p1, rs, ss,
                  _full_spec(2 * hid, 2 * out_c), _full_spec(1, 2 * out_c)],
        out_specs=[_row_spec(n, out_c)] * 3,
        out_shape=[jax.ShapeDtypeStruct((n, out_c), jnp.float32)] * 3,
    )
    decs = _tc_call(_dec_step_body, n, out_c,
                    [rs, p0, p1, rs, rs, ss,
                     _full_spec(hid + out_c, 4 * out_c),
                     _full_spec(1, 4 * out_c)], 2)

    px = agg(x[0])
    h, c = enc0(px, px, x[0], sw, W_enc, b_enc2)
    for t in range(1, t_enc):
        px = agg(x[t])
        ph = agg(h)
        h, c = encs(px, px, x[t], ph, ph, h, c, sw, W_enc, b_enc2)

    ph = agg(h)
    pc = agg(c)
    h0, c0, v = init(ph, ph, h, pc, pc, c, sw, W_init, b_init2)

    outs = jnp.zeros((t_enc, n, out_c), x.dtype)

    def dec_body(t, carry):
        h_t, c_t, acc = carry
        p = agg(h_t)
        h2, c2 = decs(v, p, p, h_t, c_t, sw, W_dec, b_dec2)
        return (h2, c2, acc.at[t].set(h2))

    _, _, outs = lax.fori_loop(0, seq_len, dec_body, (h0, c0, outs))
    return outs


# speedup vs baseline: 1.0418x; 1.0418x over previous
import functools

import jax
import jax.numpy as jnp
from jax import lax
from jax.experimental import pallas as pl
from jax.experimental.pallas import tpu as pltpu
from jax.experimental.pallas import tpu_sc as plsc

_WIN = 128
_NW = 32
_NSUB = 16


def _mesh():
    return plsc.VectorSubcoreMesh(core_axis_name="c", subcore_axis_name="s")


def _deg_call(col2, w2, n_pad):
    ep = w2.shape[0] * w2.shape[1]
    per_w = ep // _NW
    nwin = per_w // _WIN

    def body(col_h, w_h, out_h, acc, col_v, w_v, zero_v, sem):
        c = lax.axis_index("c")
        s = lax.axis_index("s")
        wid = c * _NSUB + s

        def _zero(j, _):
            zero_v[pl.ds(j * 16, 16)] = jnp.zeros((16,), jnp.float32)
            return 0

        lax.fori_loop(0, n_pad // 16, _zero, 0)

        @pl.when(s == 0)
        def _():
            pltpu.sync_copy(zero_v, acc)

        plsc.subcore_barrier()

        wbase = wid * nwin
        pltpu.sync_copy(col_h.at[pl.ds(wbase, nwin)], col_v)
        pltpu.sync_copy(w_h.at[pl.ds(wbase, nwin)], w_v)

        def win(k, _):
            pltpu.async_copy(
                w_v.at[k], acc.at[col_v.at[k]], sem, add=True)
            return 0

        lax.fori_loop(0, nwin, win, 0)

        def drain(k, _):
            pltpu.make_async_copy(
                w_v.at[0], acc.at[col_v.at[0]], sem).wait()
            return 0

        lax.fori_loop(0, nwin, drain, 0)
        plsc.subcore_barrier()

        @pl.when(s == 0)
        def _():
            pltpu.sync_copy(acc, out_h.at[c])

    return pl.kernel(
        body,
        out_type=jax.ShapeDtypeStruct((2, n_pad), jnp.float32),
        mesh=_mesh(),
        scratch_types=[
            pltpu.VMEM_SHARED((n_pad,), jnp.float32),
            pltpu.VMEM((nwin, _WIN), jnp.int32),
            pltpu.VMEM((nwin, _WIN), jnp.float32),
            pltpu.VMEM((n_pad,), jnp.float32),
            pltpu.SemaphoreType.DMA,
        ],
    )(col2, w2)


def _norm_call(row, col, w, dis):
    ep = row.shape[0]
    per_w = ep // _NW
    nwin = per_w // _WIN

    def body(row_h, col_h, w_h, dis_h, out_h, row_v, col_v, w_v,
             dr0, dr1, dc0, dc1, nv0, nv1, sg0, sg1, ss0, ss1):
        c = lax.axis_index("c")
        s = lax.axis_index("s")
        wid = c * _NSUB + s
        ebase = wid * per_w
        dr = (dr0, dr1)
        dc = (dc0, dc1)
        nv = (nv0, nv1)
        sg = (sg0, sg1)
        ss = (ss0, ss1)

        pltpu.sync_copy(row_h.at[pl.ds(ebase, per_w)], row_v)
        pltpu.sync_copy(col_h.at[pl.ds(ebase, per_w)], col_v)
        pltpu.sync_copy(w_h.at[pl.ds(ebase, per_w)], w_v)

        def issue_g(wl, b):
            pltpu.async_copy(
                dis_h.at[row_v.at[pl.ds(wl * _WIN, _WIN)]], dr[b], sg[b])
            pltpu.async_copy(
                dis_h.at[col_v.at[pl.ds(wl * _WIN, _WIN)]], dc[b], sg[b])

        def wait_g(b):
            pltpu.make_async_copy(
                dis_h.at[row_v.at[pl.ds(0, _WIN)]], dr[b], sg[b]).wait()
            pltpu.make_async_copy(
                dis_h.at[col_v.at[pl.ds(0, _WIN)]], dc[b], sg[b]).wait()

        def issue_s(wl, b):
            pltpu.async_copy(
                nv[b], out_h.at[pl.ds(ebase + wl * _WIN, _WIN)], ss[b])

        def wait_s(b):
            pltpu.make_async_copy(
                nv[b], out_h.at[pl.ds(ebase, _WIN)], ss[b]).wait()

        issue_g(0, 0)

        def pair(k0, _):
            for b in range(2):
                wl = 2 * k0 + b
                b2 = 1 - b

                @pl.when(wl >= 1)
                def _():
                    wait_s(b2)

                @pl.when(wl <= nwin - 2)
                def _():
                    issue_g(wl + 1, b2)

                wait_g(b)
                for q in range(_WIN // 16):
                    sl = pl.ds(q * 16, 16)
                    wsl = pl.ds(wl * _WIN + q * 16, 16)
                    nv[b][sl] = dr[b][sl] * w_v[wsl] * dc[b][sl]
                issue_s(wl, b)
            return 0

        lax.fori_loop(0, nwin // 2, pair, 0)
        wait_s(1)

    return pl.kernel(
        body,
        out_type=jax.ShapeDtypeStruct((ep,), jnp.float32),
        mesh=_mesh(),
        scratch_types=[
            pltpu.VMEM((per_w,), jnp.int32),
            pltpu.VMEM((per_w,), jnp.int32),
            pltpu.VMEM((per_w,), jnp.float32),
            pltpu.VMEM((_WIN,), jnp.float32),
            pltpu.VMEM((_WIN,), jnp.float32),
            pltpu.VMEM((_WIN,), jnp.float32),
            pltpu.VMEM((_WIN,), jnp.float32),
            pltpu.VMEM((_WIN,), jnp.float32),
            pltpu.VMEM((_WIN,), jnp.float32),
            pltpu.SemaphoreType.DMA,
            pltpu.SemaphoreType.DMA,
            pltpu.SemaphoreType.DMA,
            pltpu.SemaphoreType.DMA,
        ],
    )(row, col, w, dis)


def _agg_call(table, row, col2, norm, n_pad):
    n_nodes, ch = table.shape
    ep = row.shape[0]
    per_w = ep // _NW
    nwin = per_w // _WIN
    nseg = 2
    half = nwin // nseg
    hlen = half * _WIN
    rows_per_sub = n_pad // _NSUB

    zfull = rows_per_sub // _WIN
    zrem = rows_per_sub % _WIN

    def body(table_h, row_h, col_h, norm_h, out_h, part, row_v, col_v,
             norm_v, rows_v0, rows_v1, sg0, sg1, ss0, ss1):
        c = lax.axis_index("c")
        s = lax.axis_index("s")
        wid = c * _NSUB + s
        rows_b = (rows_v0, rows_v1)
        sg = (sg0, sg1)
        ss = (ss0, ss1)

        def _zero(j, _):
            for q in range(ch // 16):
                rows_v0[j, pl.ds(q * 16, 16)] = jnp.zeros((16,), jnp.float32)
            return 0

        lax.fori_loop(0, _WIN, _zero, 0)
        for k in range(zfull):
            pltpu.sync_copy(
                rows_v0, part.at[pl.ds(s * rows_per_sub + k * _WIN, _WIN)])
        if zrem:
            pltpu.sync_copy(
                rows_v0.at[pl.ds(0, zrem)],
                part.at[pl.ds(s * rows_per_sub + zfull * _WIN, zrem)])
        plsc.subcore_barrier()

        def issue_g(wl, b):
            return pltpu.async_copy(
                table_h.at[row_v.at[pl.ds(wl * _WIN, _WIN)]], rows_b[b], sg[b])

        def wait_g(b):
            pltpu.make_async_copy(
                table_h.at[row_v.at[pl.ds(0, _WIN)]], rows_b[b], sg[b]).wait()

        def issue_s(wl, b):
            return pltpu.async_copy(
                rows_b[b], part.at[col_v.at[wl]], ss[b], add=True)

        def wait_s(b):
            pltpu.make_async_copy(
                rows_b[b], part.at[col_v.at[0]], ss[b]).wait()

        def do_mul(wl, b):
            rv = rows_b[b]

            @plsc.parallel_loop(0, _WIN // 16, unroll=2)
            def mul16(kb):
                nv16 = norm_v[pl.ds(wl * _WIN + kb * 16, 16)]
                for j in range(16):
                    nv = jnp.full((16,), nv16[j], jnp.float32)
                    e = kb * 16 + j
                    for q in range(ch // 16):
                        sl = pl.ds(q * 16, 16)
                        rv[e, sl] = rv[e, sl] * nv

            issue_s(wl, b)

        for h in range(nseg):
            ebase = wid * per_w + h * hlen
            wbase = wid * nwin + h * half
            pltpu.sync_copy(row_h.at[pl.ds(ebase, hlen)], row_v)
            pltpu.sync_copy(col_h.at[pl.ds(wbase, half)], col_v)
            pltpu.sync_copy(norm_h.at[pl.ds(ebase, hlen)], norm_v)
            issue_g(0, 0)

            def pair(k0, _):
                for b in range(2):
                    wl = 2 * k0 + b
                    b2 = 1 - b

                    @pl.when(wl >= 1)
                    def _():
                        wait_s(b2)

                    @pl.when(wl <= half - 2)
                    def _():
                        issue_g(wl + 1, b2)

                    wait_g(b)
                    do_mul(wl, b)
                return 0

            lax.fori_loop(0, half // 2, pair, 0)
            wait_s(1)

        plsc.subcore_barrier()
        pltpu.sync_copy(
            part.at[pl.ds(s * rows_per_sub, rows_per_sub)],
            out_h.at[c].at[pl.ds(s * rows_per_sub, rows_per_sub)])

    return pl.kernel(
        body,
        out_type=jax.ShapeDtypeStruct((2, n_pad, ch), jnp.float32),
        mesh=_mesh(),
        scratch_types=[
            pltpu.VMEM_SHARED((n_pad, ch), jnp.float32),
            pltpu.VMEM((hlen,), jnp.int32),
            pltpu.VMEM((half, _WIN), jnp.int32),
            pltpu.VMEM((hlen,), jnp.float32),
            pltpu.VMEM((_WIN, ch), jnp.float32),
            pltpu.VMEM((_WIN, ch), jnp.float32),
            pltpu.SemaphoreType.DMA,
            pltpu.SemaphoreType.DMA,
            pltpu.SemaphoreType.DMA,
            pltpu.SemaphoreType.DMA,
        ],
    )(table, row, col2, norm)


_GRID = 10


def _lstm_tail(cc, c_prev, hid):
    i = jax.nn.sigmoid(cc[:, :hid])
    f = jax.nn.sigmoid(cc[:, hid:2 * hid])
    o = jax.nn.sigmoid(cc[:, 2 * hid:3 * hid])
    g = jnp.tanh(cc[:, 3 * hid:])
    c2 = f * c_prev + i * g
    return o * jnp.tanh(c2), c2


def _enc_step_body(px0, px1, xt, ph0, ph1, hp, cp, sw, w, b, h_out, c_out):
    hid = h_out.shape[1]
    cin = xt.shape[1]
    wf = w[...]
    aggx = px0[0] + px1[0] + sw[...] * xt[...]
    aggh = ph0[0] + ph1[0] + sw[...] * hp[...]
    cc = (jnp.dot(aggx, wf[:cin], preferred_element_type=jnp.float32)
          + jnp.dot(aggh, wf[cin:], preferred_element_type=jnp.float32)
          + b[...])
    h2, c2 = _lstm_tail(cc, cp[...], hid)
    h_out[...] = h2
    c_out[...] = c2


def _enc_step0_body(px0, px1, xt, sw, w, b, h_out, c_out):
    hid = h_out.shape[1]
    cin = xt.shape[1]
    wf = w[...]
    aggx = px0[0] + px1[0] + sw[...] * xt[...]
    cc = jnp.dot(aggx, wf[:cin], preferred_element_type=jnp.float32) + b[...]
    i = jax.nn.sigmoid(cc[:, :hid])
    o = jax.nn.sigmoid(cc[:, 2 * hid:3 * hid])
    g = jnp.tanh(cc[:, 3 * hid:])
    c2 = i * g
    h_out[...] = o * jnp.tanh(c2)
    c_out[...] = c2


def _dec_step_body(vx, ph0, ph1, hp, cp, sw, w, b, h_out, c_out):
    hid = h_out.shape[1]
    wf = w[...]
    aggh = ph0[0] + ph1[0] + sw[...] * hp[...]
    cc = (jnp.dot(vx[...], wf[:hid], preferred_element_type=jnp.float32)
          + jnp.dot(aggh, wf[hid:], preferred_element_type=jnp.float32)
          + b[...])
    h2, c2 = _lstm_tail(cc, cp[...], hid)
    h_out[...] = h2
    c_out[...] = c2


def _init_body(ph0, ph1, hp, pc0, pc1, cp, sw, w, b, h_out, c_out, v_out):
    hid = h_out.shape[1]
    wf = w[...]
    aggh = ph0[0] + ph1[0] + sw[...] * hp[...]
    aggc = pc0[0] + pc1[0] + sw[...] * cp[...]
    st = (jnp.dot(aggh, wf[:hid], preferred_element_type=jnp.float32)
          + jnp.dot(aggc, wf[hid:], preferred_element_type=jnp.float32)
          + b[...])
    st = jnp.where(st > 0, st, jnp.exp(jnp.minimum(st, 0.0)) - 1.0)
    h_out[...] = st[:, :hid]
    c_out[...] = st[:, hid:]
    v_out[...] = aggh


def _row_spec(n, ch):
    return pl.BlockSpec((n // _GRID, ch), lambda i: (i, 0))


def _full_spec(r, c):
    return pl.BlockSpec((r, c), lambda i: (0, 0))


def _tc_call(body, n, hid, in_specs, num_outs):
    return pl.pallas_call(
        body,
        grid=(_GRID,),
        in_specs=in_specs,
        out_specs=[_row_spec(n, hid)] * num_outs,
        out_shape=[jax.ShapeDtypeStruct((n, hid), jnp.float32)] * num_outs,
    )


def _dis_body(deg, dis_out, sw_out):
    d = deg[0:1, :] + deg[1:2, :] + 1.0
    dis_out[...] = lax.rsqrt(d)
    sw_out[...] = 1.0 / d


def kernel(x, edge_index, edge_attr, seq_len, W_enc, b_enc, W_init, b_init,
           W_dec, b_dec):
    t_enc, n, cin = x.shape
    hid = W_enc.shape[1] // 4
    out_c = W_dec.shape[1] // 4
    e = edge_index.shape[1]

    chunk = _NW * _WIN * 2
    ep = ((e + chunk - 1) // chunk) * chunk
    pad = ep - e
    row = edge_index[0]
    col = edge_index[1]
    w = edge_attr
    if pad:
        fill = (jnp.arange(pad, dtype=jnp.int32) * 97) % n
        row = jnp.concatenate([row, fill])
        col = jnp.concatenate([col, fill])
        w = jnp.concatenate([w, jnp.zeros((pad,), jnp.float32)])

    n_pad = ((n + 127) // 128) * 128

    col2 = col.reshape(-1, _WIN)
    degp = _deg_call(col2, w.reshape(-1, _WIN), n_pad)
    dis2, sw2 = pl.pallas_call(
        _dis_body,
        in_specs=[pl.BlockSpec((2, n_pad), lambda: (0, 0))],
        out_specs=[pl.BlockSpec((1, n_pad), lambda: (0, 0))] * 2,
        out_shape=[jax.ShapeDtypeStruct((1, n_pad), jnp.float32)] * 2,
    )(degp)
    dis = dis2.reshape(n_pad)
    sw = sw2.reshape(n_pad, 1)
    norm = _norm_call(row, col, w, dis)

    agg = lambda tbl: _agg_call(tbl, row, col2, norm, n_pad)

    b_enc2 = b_enc.reshape(1, -1)
    b_init2 = b_init.reshape(1, -1)
    b_dec2 = b_dec.reshape(1, -1)

    r = n // _GRID
    rs = _row_spec(n, hid)
    ss = pl.BlockSpec((r, 1), lambda i: (i, 0))
    p0 = pl.BlockSpec((1, r, hid), lambda i: (0, i, 0))
    p1 = pl.BlockSpec((1, r, hid), lambda i: (1, i, 0))
    wspec_enc = _full_spec(cin + hid, 4 * hid)
    bspec_enc = _full_spec(1, 4 * hid)

    enc0 = _tc_call(_enc_step0_body, n, hid,
                    [p0, p1, rs, ss, wspec_enc, bspec_enc], 2)
    encs = _tc_call(_enc_step_body, n, hid,
                    [p0, p1, rs, p0, p1, rs, rs, ss, wspec_enc, bspec_enc], 2)
    init = pl.pallas_call(
        _init_body,
        grid=(_GRID,),
        in_specs=[p0, p1, rs, p0, p1, rs, ss,
                  _full_spec(2 * hid, 2 * out_c), _full_spec(1, 2 * out_c)],
        out_specs=[_row_spec(n, out_c)] * 3,
        out_shape=[jax.ShapeDtypeStruct((n, out_c), jnp.float32)] * 3,
    )
    decs = _tc_call(_dec_step_body, n, out_c,
                    [rs, p0, p1, rs, rs, ss,
                     _full_spec(hid + out_c, 4 * out_c),
                     _full_spec(1, 4 * out_c)], 2)

    px = agg(x[0])
    h, c = enc0(px, px, x[0], sw, W_enc, b_enc2)
    for t in range(1, t_enc):
        px = agg(x[t])
        ph = agg(h)
        h, c = encs(px, px, x[t], ph, ph, h, c, sw, W_enc, b_enc2)

    ph = agg(h)
    pc = agg(c)
    h0, c0, v = init(ph, ph, h, pc, pc, c, sw, W_init, b_init2)

    outs = jnp.zeros((t_enc, n, out_c), x.dtype)

    def dec_body(t, carry):
        h_t, c_t, acc = carry
        p = agg(h_t)
        h2, c2 = decs(v, p, p, h_t, c_t, sw, W_dec, b_dec2)
        return (h2, c2, acc.at[t].set(h2))

    _, _, outs = lax.fori_loop(0, seq_len, dec_body, (h0, c0, outs))
    return outs


# speedup vs baseline: 1.0525x; 1.0102x over previous
import functools

import jax
import jax.numpy as jnp
from jax import lax
from jax.experimental import pallas as pl
from jax.experimental.pallas import tpu as pltpu
from jax.experimental.pallas import tpu_sc as plsc

_WIN = 128
_NW = 32
_NSUB = 16


def _mesh():
    return plsc.VectorSubcoreMesh(core_axis_name="c", subcore_axis_name="s")


def _deg_call(col2, w2, n_pad):
    ep = w2.shape[0] * w2.shape[1]
    per_w = ep // _NW
    nwin = per_w // _WIN

    def body(col_h, w_h, out_h, acc, col_v, w_v, zero_v, sem):
        c = lax.axis_index("c")
        s = lax.axis_index("s")
        wid = c * _NSUB + s

        def _zero(j, _):
            zero_v[pl.ds(j * 16, 16)] = jnp.zeros((16,), jnp.float32)
            return 0

        lax.fori_loop(0, n_pad // 16, _zero, 0)

        @pl.when(s == 0)
        def _():
            pltpu.sync_copy(zero_v, acc)

        plsc.subcore_barrier()

        wbase = wid * nwin
        pltpu.sync_copy(col_h.at[pl.ds(wbase, nwin)], col_v)
        pltpu.sync_copy(w_h.at[pl.ds(wbase, nwin)], w_v)

        def win(k, _):
            pltpu.async_copy(
                w_v.at[k], acc.at[col_v.at[k]], sem, add=True)
            return 0

        lax.fori_loop(0, nwin, win, 0)

        def drain(k, _):
            pltpu.make_async_copy(
                w_v.at[0], acc.at[col_v.at[0]], sem).wait()
            return 0

        lax.fori_loop(0, nwin, drain, 0)
        plsc.subcore_barrier()

        @pl.when(s == 0)
        def _():
            pltpu.sync_copy(acc, out_h.at[c])

    return pl.kernel(
        body,
        out_type=jax.ShapeDtypeStruct((2, n_pad), jnp.float32),
        mesh=_mesh(),
        scratch_types=[
            pltpu.VMEM_SHARED((n_pad,), jnp.float32),
            pltpu.VMEM((nwin, _WIN), jnp.int32),
            pltpu.VMEM((nwin, _WIN), jnp.float32),
            pltpu.VMEM((n_pad,), jnp.float32),
            pltpu.SemaphoreType.DMA,
        ],
    )(col2, w2)


def _norm_call(row, col, w, dis):
    ep = row.shape[0]
    per_w = ep // _NW
    nwin = per_w // _WIN

    def body(row_h, col_h, w_h, dis_h, out_h, row_v, col_v, w_v,
             dr0, dr1, dc0, dc1, nv0, nv1, sg0, sg1, ss0, ss1):
        c = lax.axis_index("c")
        s = lax.axis_index("s")
        wid = c * _NSUB + s
        ebase = wid * per_w
        dr = (dr0, dr1)
        dc = (dc0, dc1)
        nv = (nv0, nv1)
        sg = (sg0, sg1)
        ss = (ss0, ss1)

        pltpu.sync_copy(row_h.at[pl.ds(ebase, per_w)], row_v)
        pltpu.sync_copy(col_h.at[pl.ds(ebase, per_w)], col_v)
        pltpu.sync_copy(w_h.at[pl.ds(ebase, per_w)], w_v)

        def issue_g(wl, b):
            pltpu.async_copy(
                dis_h.at[row_v.at[pl.ds(wl * _WIN, _WIN)]], dr[b], sg[b])
            pltpu.async_copy(
                dis_h.at[col_v.at[pl.ds(wl * _WIN, _WIN)]], dc[b], sg[b])

        def wait_g(b):
            pltpu.make_async_copy(
                dis_h.at[row_v.at[pl.ds(0, _WIN)]], dr[b], sg[b]).wait()
            pltpu.make_async_copy(
                dis_h.at[col_v.at[pl.ds(0, _WIN)]], dc[b], sg[b]).wait()

        def issue_s(wl, b):
            pltpu.async_copy(
                nv[b], out_h.at[pl.ds(ebase + wl * _WIN, _WIN)], ss[b])

        def wait_s(b):
            pltpu.make_async_copy(
                nv[b], out_h.at[pl.ds(ebase, _WIN)], ss[b]).wait()

        issue_g(0, 0)

        def pair(k0, _):
            for b in range(2):
                wl = 2 * k0 + b
                b2 = 1 - b

                @pl.when(wl >= 1)
                def _():
                    wait_s(b2)

                @pl.when(wl <= nwin - 2)
                def _():
                    issue_g(wl + 1, b2)

                wait_g(b)
                for q in range(_WIN // 16):
                    sl = pl.ds(q * 16, 16)
                    wsl = pl.ds(wl * _WIN + q * 16, 16)
                    nv[b][sl] = dr[b][sl] * w_v[wsl] * dc[b][sl]
                issue_s(wl, b)
            return 0

        lax.fori_loop(0, nwin // 2, pair, 0)
        wait_s(1)

    return pl.kernel(
        body,
        out_type=jax.ShapeDtypeStruct((ep,), jnp.float32),
        mesh=_mesh(),
        scratch_types=[
            pltpu.VMEM((per_w,), jnp.int32),
            pltpu.VMEM((per_w,), jnp.int32),
            pltpu.VMEM((per_w,), jnp.float32),
            pltpu.VMEM((_WIN,), jnp.float32),
            pltpu.VMEM((_WIN,), jnp.float32),
            pltpu.VMEM((_WIN,), jnp.float32),
            pltpu.VMEM((_WIN,), jnp.float32),
            pltpu.VMEM((_WIN,), jnp.float32),
            pltpu.VMEM((_WIN,), jnp.float32),
            pltpu.SemaphoreType.DMA,
            pltpu.SemaphoreType.DMA,
            pltpu.SemaphoreType.DMA,
            pltpu.SemaphoreType.DMA,
        ],
    )(row, col, w, dis)


def _agg_call(table, row, col2, norm, n_pad):
    n_nodes, ch = table.shape
    ep = row.shape[0]
    per_w = ep // _NW
    nwin = per_w // _WIN
    nseg = 2
    half = nwin // nseg
    hlen = half * _WIN
    rows_per_sub = n_pad // _NSUB

    zfull = rows_per_sub // _WIN
    zrem = rows_per_sub % _WIN

    def body(table_h, row_h, col_h, norm_h, out_h, part, row_v, col_v,
             norm_v, rows_v0, rows_v1, sg0, sg1, ss0, ss1):
        c = lax.axis_index("c")
        s = lax.axis_index("s")
        wid = c * _NSUB + s
        rows_b = (rows_v0, rows_v1)
        sg = (sg0, sg1)
        ss = (ss0, ss1)

        def _zero(j, _):
            for q in range(ch // 16):
                rows_v0[j, pl.ds(q * 16, 16)] = jnp.zeros((16,), jnp.float32)
            return 0

        lax.fori_loop(0, _WIN, _zero, 0)
        for k in range(zfull):
            pltpu.sync_copy(
                rows_v0, part.at[pl.ds(s * rows_per_sub + k * _WIN, _WIN)])
        if zrem:
            pltpu.sync_copy(
                rows_v0.at[pl.ds(0, zrem)],
                part.at[pl.ds(s * rows_per_sub + zfull * _WIN, zrem)])
        plsc.subcore_barrier()

        def issue_g(wl, b):
            return pltpu.async_copy(
                table_h.at[row_v.at[pl.ds(wl * _WIN, _WIN)]], rows_b[b], sg[b])

        def wait_g(b):
            pltpu.make_async_copy(
                table_h.at[row_v.at[pl.ds(0, _WIN)]], rows_b[b], sg[b]).wait()

        def issue_s(wl, b):
            return pltpu.async_copy(
                rows_b[b], part.at[col_v.at[wl]], ss[b], add=True)

        def wait_s(b):
            pltpu.make_async_copy(
                rows_b[b], part.at[col_v.at[0]], ss[b]).wait()

        def do_mul(wl, b):
            rv = rows_b[b]

            def mul16(kb, _):
                nv16 = norm_v[pl.ds(wl * _WIN + kb * 16, 16)]
                for j in range(16):
                    nv = jnp.full((16,), nv16[j], jnp.float32)
                    e = kb * 16 + j
                    for q in range(ch // 16):
                        sl = pl.ds(q * 16, 16)
                        rv[e, sl] = rv[e, sl] * nv
                return 0

            lax.fori_loop(0, _WIN // 16, mul16, 0)
            issue_s(wl, b)

        for h in range(nseg):
            ebase = wid * per_w + h * hlen
            wbase = wid * nwin + h * half
            pltpu.sync_copy(row_h.at[pl.ds(ebase, hlen)], row_v)
            pltpu.sync_copy(col_h.at[pl.ds(wbase, half)], col_v)
            pltpu.sync_copy(norm_h.at[pl.ds(ebase, hlen)], norm_v)
            issue_g(0, 0)

            def pair(k0, _):
                for b in range(2):
                    wl = 2 * k0 + b
                    b2 = 1 - b

                    @pl.when(wl >= 1)
                    def _():
                        wait_s(b2)

                    @pl.when(wl <= half - 2)
                    def _():
                        issue_g(wl + 1, b2)

                    wait_g(b)
                    do_mul(wl, b)
                return 0

            lax.fori_loop(0, half // 2, pair, 0)
            wait_s(1)

        plsc.subcore_barrier()
        pltpu.sync_copy(
            part.at[pl.ds(s * rows_per_sub, rows_per_sub)],
            out_h.at[c].at[pl.ds(s * rows_per_sub, rows_per_sub)])

    return pl.kernel(
        body,
        out_type=jax.ShapeDtypeStruct((2, n_pad, ch), jnp.float32),
        mesh=_mesh(),
        scratch_types=[
            pltpu.VMEM_SHARED((n_pad, ch), jnp.float32),
            pltpu.VMEM((hlen,), jnp.int32),
            pltpu.VMEM((half, _WIN), jnp.int32),
            pltpu.VMEM((hlen,), jnp.float32),
            pltpu.VMEM((_WIN, ch), jnp.float32),
            pltpu.VMEM((_WIN, ch), jnp.float32),
            pltpu.SemaphoreType.DMA,
            pltpu.SemaphoreType.DMA,
            pltpu.SemaphoreType.DMA,
            pltpu.SemaphoreType.DMA,
        ],
    )(table, row, col2, norm)


_GRID = 10


def _lstm_tail(cc, c_prev, hid):
    i = jax.nn.sigmoid(cc[:, :hid])
    f = jax.nn.sigmoid(cc[:, hid:2 * hid])
    o = jax.nn.sigmoid(cc[:, 2 * hid:3 * hid])
    g = jnp.tanh(cc[:, 3 * hid:])
    c2 = f * c_prev + i * g
    return o * jnp.tanh(c2), c2


def _enc_step_body(px0, px1, xt, ph0, ph1, hp, cp, sw, w, b, h_out, c_out):
    hid = h_out.shape[1]
    cin = xt.shape[1]
    wf = w[...]
    aggx = px0[0] + px1[0] + sw[...] * xt[...]
    aggh = ph0[0] + ph1[0] + sw[...] * hp[...]
    cc = (jnp.dot(aggx, wf[:cin], preferred_element_type=jnp.float32)
          + jnp.dot(aggh, wf[cin:], preferred_element_type=jnp.float32)
          + b[...])
    h2, c2 = _lstm_tail(cc, cp[...], hid)
    h_out[...] = h2
    c_out[...] = c2


def _enc_step0_body(px0, px1, xt, sw, w, b, h_out, c_out):
    hid = h_out.shape[1]
    cin = xt.shape[1]
    wf = w[...]
    aggx = px0[0] + px1[0] + sw[...] * xt[...]
    cc = jnp.dot(aggx, wf[:cin], preferred_element_type=jnp.float32) + b[...]
    i = jax.nn.sigmoid(cc[:, :hid])
    o = jax.nn.sigmoid(cc[:, 2 * hid:3 * hid])
    g = jnp.tanh(cc[:, 3 * hid:])
    c2 = i * g
    h_out[...] = o * jnp.tanh(c2)
    c_out[...] = c2


def _dec_step_body(vx, ph0, ph1, hp, cp, sw, w, b, h_out, c_out):
    hid = h_out.shape[1]
    wf = w[...]
    aggh = ph0[0] + ph1[0] + sw[...] * hp[...]
    cc = (jnp.dot(vx[...], wf[:hid], preferred_element_type=jnp.float32)
          + jnp.dot(aggh, wf[hid:], preferred_element_type=jnp.float32)
          + b[...])
    h2, c2 = _lstm_tail(cc, cp[...], hid)
    h_out[...] = h2
    c_out[...] = c2


def _init_body(ph0, ph1, hp, pc0, pc1, cp, sw, w, b, h_out, c_out, v_out):
    hid = h_out.shape[1]
    wf = w[...]
    aggh = ph0[0] + ph1[0] + sw[...] * hp[...]
    aggc = pc0[0] + pc1[0] + sw[...] * cp[...]
    st = (jnp.dot(aggh, wf[:hid], preferred_element_type=jnp.float32)
          + jnp.dot(aggc, wf[hid:], preferred_element_type=jnp.float32)
          + b[...])
    st = jnp.where(st > 0, st, jnp.exp(jnp.minimum(st, 0.0)) - 1.0)
    h_out[...] = st[:, :hid]
    c_out[...] = st[:, hid:]
    v_out[...] = aggh


def _row_spec(n, ch):
    return pl.BlockSpec((n // _GRID, ch), lambda i: (i, 0))


def _full_spec(r, c):
    return pl.BlockSpec((r, c), lambda i: (0, 0))


def _tc_call(body, n, hid, in_specs, num_outs):
    return pl.pallas_call(
        body,
        grid=(_GRID,),
        in_specs=in_specs,
        out_specs=[_row_spec(n, hid)] * num_outs,
        out_shape=[jax.ShapeDtypeStruct((n, hid), jnp.float32)] * num_outs,
    )


def _dis_body(deg, dis_out, sw_out):
    d = deg[0:1, :] + deg[1:2, :] + 1.0
    dis_out[...] = lax.rsqrt(d)
    sw_out[...] = 1.0 / d


def kernel(x, edge_index, edge_attr, seq_len, W_enc, b_enc, W_init, b_init,
           W_dec, b_dec):
    t_enc, n, cin = x.shape
    hid = W_enc.shape[1] // 4
    out_c = W_dec.shape[1] // 4
    e = edge_index.shape[1]

    chunk = _NW * _WIN * 2
    ep = ((e + chunk - 1) // chunk) * chunk
    pad = ep - e
    row = edge_index[0]
    col = edge_index[1]
    w = edge_attr
    if pad:
        fill = (jnp.arange(pad, dtype=jnp.int32) * 97) % n
        row = jnp.concatenate([row, fill])
        col = jnp.concatenate([col, fill])
        w = jnp.concatenate([w, jnp.zeros((pad,), jnp.float32)])

    n_pad = ((n + 127) // 128) * 128

    col2 = col.reshape(-1, _WIN)
    degp = _deg_call(col2, w.reshape(-1, _WIN), n_pad)
    dis2, sw2 = pl.pallas_call(
        _dis_body,
        in_specs=[pl.BlockSpec((2, n_pad), lambda: (0, 0))],
        out_specs=[pl.BlockSpec((1, n_pad), lambda: (0, 0))] * 2,
        out_shape=[jax.ShapeDtypeStruct((1, n_pad), jnp.float32)] * 2,
    )(degp)
    dis = dis2.reshape(n_pad)
    sw = sw2.reshape(n_pad, 1)
    norm = _norm_call(row, col, w, dis)

    agg = lambda tbl: _agg_call(tbl, row, col2, norm, n_pad)

    b_enc2 = b_enc.reshape(1, -1)
    b_init2 = b_init.reshape(1, -1)
    b_dec2 = b_dec.reshape(1, -1)

    r = n // _GRID
    rs = _row_spec(n, hid)
    ss = pl.BlockSpec((r, 1), lambda i: (i, 0))
    p0 = pl.BlockSpec((1, r, hid), lambda i: (0, i, 0))
    p1 = pl.BlockSpec((1, r, hid), lambda i: (1, i, 0))
    wspec_enc = _full_spec(cin + hid, 4 * hid)
    bspec_enc = _full_spec(1, 4 * hid)

    enc0 = _tc_call(_enc_step0_body, n, hid,
                    [p0, p1, rs, ss, wspec_enc, bspec_enc], 2)
    encs = _tc_call(_enc_step_body, n, hid,
                    [p0, p1, rs, p0, p1, rs, rs, ss, wspec_enc, bspec_enc], 2)
    init = pl.pallas_call(
        _init_body,
        grid=(_GRID,),
        in_specs=[p0, p1, rs, p0, p1, rs, ss,
                  _full_spec(2 * hid, 2 * out_c), _full_spec(1, 2 * out_c)],
        out_specs=[_row_spec(n, out_c)] * 3,
        out_shape=[jax.ShapeDtypeStruct((n, out_c), jnp.float32)] * 3,
    )
    decs = _tc_call(_dec_step_body, n, out_c,
                    [rs, p0, p1, rs, rs, ss,
                     _full_spec(hid + out_c, 4 * out_c),
                     _full_spec(1, 4 * out_c)], 2)

    px = agg(x[0])
    h, c = enc0(px, px, x[0], sw, W_enc, b_enc2)
    for t in range(1, t_enc):
        px = agg(x[t])
        ph = agg(h)
        h, c = encs(px, px, x[t], ph, ph, h, c, sw, W_enc, b_enc2)

    ph = agg(h)
    pc = agg(c)
    h0, c0, v = init(ph, ph, h, pc, pc, c, sw, W_init, b_init2)

    outs = jnp.zeros((t_enc, n, out_c), x.dtype)

    def dec_body(t, carry):
        h_t, c_t, acc = carry
        p = agg(h_t)
        h2, c2 = decs(v, p, p, h_t, c_t, sw, W_dec, b_dec2)
        return (h2, c2, acc.at[t].set(h2))

    _, _, outs = lax.fori_loop(0, seq_len, dec_body, (h0, c0, outs))
    return outs


# speedup vs baseline: 1.0593x; 1.0065x over previous
import functools

import jax
import jax.numpy as jnp
from jax import lax
from jax.experimental import pallas as pl
from jax.experimental.pallas import tpu as pltpu
from jax.experimental.pallas import tpu_sc as plsc

_WIN = 128
_NW = 32
_NSUB = 16


def _mesh():
    return plsc.VectorSubcoreMesh(core_axis_name="c", subcore_axis_name="s")


def _deg_call(col2, w2, n_pad):
    ep = w2.shape[0] * w2.shape[1]
    per_w = ep // _NW
    nwin = per_w // _WIN

    def body(col_h, w_h, out_h, acc, col_v, w_v, zero_v, sem):
        c = lax.axis_index("c")
        s = lax.axis_index("s")
        wid = c * _NSUB + s

        def _zero(j, _):
            zero_v[pl.ds(j * 16, 16)] = jnp.zeros((16,), jnp.float32)
            return 0

        lax.fori_loop(0, n_pad // 16, _zero, 0)

        @pl.when(s == 0)
        def _():
            pltpu.sync_copy(zero_v, acc)

        plsc.subcore_barrier()

        wbase = wid * nwin
        pltpu.sync_copy(col_h.at[pl.ds(wbase, nwin)], col_v)
        pltpu.sync_copy(w_h.at[pl.ds(wbase, nwin)], w_v)

        def win(k, _):
            pltpu.async_copy(
                w_v.at[k], acc.at[col_v.at[k]], sem, add=True)
            return 0

        lax.fori_loop(0, nwin, win, 0)

        def drain(k, _):
            pltpu.make_async_copy(
                w_v.at[0], acc.at[col_v.at[0]], sem).wait()
            return 0

        lax.fori_loop(0, nwin, drain, 0)
        plsc.subcore_barrier()

        @pl.when(s == 0)
        def _():
            pltpu.sync_copy(acc, out_h.at[c])

    return pl.kernel(
        body,
        out_type=jax.ShapeDtypeStruct((2, n_pad), jnp.float32),
        mesh=_mesh(),
        scratch_types=[
            pltpu.VMEM_SHARED((n_pad,), jnp.float32),
            pltpu.VMEM((nwin, _WIN), jnp.int32),
            pltpu.VMEM((nwin, _WIN), jnp.float32),
            pltpu.VMEM((n_pad,), jnp.float32),
            pltpu.SemaphoreType.DMA,
        ],
    )(col2, w2)


def _norm_call(row, col, w, dis):
    ep = row.shape[0]
    per_w = ep // _NW
    nwin = per_w // _WIN

    def body(row_h, col_h, w_h, dis_h, out_h, row_v, col_v, w_v,
             dr0, dr1, dc0, dc1, nv0, nv1, sg0, sg1, ss0, ss1):
        c = lax.axis_index("c")
        s = lax.axis_index("s")
        wid = c * _NSUB + s
        ebase = wid * per_w
        dr = (dr0, dr1)
        dc = (dc0, dc1)
        nv = (nv0, nv1)
        sg = (sg0, sg1)
        ss = (ss0, ss1)

        pltpu.sync_copy(row_h.at[pl.ds(ebase, per_w)], row_v)
        pltpu.sync_copy(col_h.at[pl.ds(ebase, per_w)], col_v)
        pltpu.sync_copy(w_h.at[pl.ds(ebase, per_w)], w_v)

        def issue_g(wl, b):
            pltpu.async_copy(
                dis_h.at[row_v.at[pl.ds(wl * _WIN, _WIN)]], dr[b], sg[b])
            pltpu.async_copy(
                dis_h.at[col_v.at[pl.ds(wl * _WIN, _WIN)]], dc[b], sg[b])

        def wait_g(b):
            pltpu.make_async_copy(
                dis_h.at[row_v.at[pl.ds(0, _WIN)]], dr[b], sg[b]).wait()
            pltpu.make_async_copy(
                dis_h.at[col_v.at[pl.ds(0, _WIN)]], dc[b], sg[b]).wait()

        def issue_s(wl, b):
            pltpu.async_copy(
                nv[b], out_h.at[pl.ds(ebase + wl * _WIN, _WIN)], ss[b])

        def wait_s(b):
            pltpu.make_async_copy(
                nv[b], out_h.at[pl.ds(ebase, _WIN)], ss[b]).wait()

        issue_g(0, 0)

        def pair(k0, _):
            for b in range(2):
                wl = 2 * k0 + b
                b2 = 1 - b

                @pl.when(wl >= 1)
                def _():
                    wait_s(b2)

                @pl.when(wl <= nwin - 2)
                def _():
                    issue_g(wl + 1, b2)

                wait_g(b)
                for q in range(_WIN // 16):
                    sl = pl.ds(q * 16, 16)
                    wsl = pl.ds(wl * _WIN + q * 16, 16)
                    nv[b][sl] = dr[b][sl] * w_v[wsl] * dc[b][sl]
                issue_s(wl, b)
            return 0

        lax.fori_loop(0, nwin // 2, pair, 0)
        wait_s(1)

    return pl.kernel(
        body,
        out_type=jax.ShapeDtypeStruct((ep,), jnp.float32),
        mesh=_mesh(),
        scratch_types=[
            pltpu.VMEM((per_w,), jnp.int32),
            pltpu.VMEM((per_w,), jnp.int32),
            pltpu.VMEM((per_w,), jnp.float32),
            pltpu.VMEM((_WIN,), jnp.float32),
            pltpu.VMEM((_WIN,), jnp.float32),
            pltpu.VMEM((_WIN,), jnp.float32),
            pltpu.VMEM((_WIN,), jnp.float32),
            pltpu.VMEM((_WIN,), jnp.float32),
            pltpu.VMEM((_WIN,), jnp.float32),
            pltpu.SemaphoreType.DMA,
            pltpu.SemaphoreType.DMA,
            pltpu.SemaphoreType.DMA,
            pltpu.SemaphoreType.DMA,
        ],
    )(row, col, w, dis)


def _agg_call(table, row, col2, norm, n_pad):
    n_nodes, ch = table.shape
    ep = row.shape[0]
    per_w = ep // _NW
    nwin = per_w // _WIN
    nseg = 2
    half = nwin // nseg
    hlen = half * _WIN
    rows_per_sub = n_pad // _NSUB

    zfull = rows_per_sub // _WIN
    zrem = rows_per_sub % _WIN

    def body(table_h, row_h, col_h, norm_h, out_h, part, row_v, col_v,
             norm_v, rows_v0, rows_v1, sg0, sg1, ss0, ss1):
        c = lax.axis_index("c")
        s = lax.axis_index("s")
        wid = c * _NSUB + s
        rows_b = (rows_v0, rows_v1)
        sg = (sg0, sg1)
        ss = (ss0, ss1)

        def _zero(j, _):
            for q in range(ch // 16):
                rows_v0[j, pl.ds(q * 16, 16)] = jnp.zeros((16,), jnp.float32)
            return 0

        lax.fori_loop(0, _WIN, _zero, 0)
        for k in range(zfull):
            pltpu.sync_copy(
                rows_v0, part.at[pl.ds(s * rows_per_sub + k * _WIN, _WIN)])
        if zrem:
            pltpu.sync_copy(
                rows_v0.at[pl.ds(0, zrem)],
                part.at[pl.ds(s * rows_per_sub + zfull * _WIN, zrem)])
        plsc.subcore_barrier()

        def issue_g(wl, b):
            return pltpu.async_copy(
                table_h.at[row_v.at[pl.ds(wl * _WIN, _WIN)]], rows_b[b], sg[b])

        def wait_g(b):
            pltpu.make_async_copy(
                table_h.at[row_v.at[pl.ds(0, _WIN)]], rows_b[b], sg[b]).wait()

        def issue_s(wl, b):
            return pltpu.async_copy(
                rows_b[b], part.at[col_v.at[wl]], ss[b], add=True)

        def wait_s(b):
            pltpu.make_async_copy(
                rows_b[b], part.at[col_v.at[0]], ss[b]).wait()

        def do_mul(wl, b):
            rv = rows_b[b]

            def mul16(kb, _):
                nv16 = norm_v[pl.ds(wl * _WIN + kb * 16, 16)]
                for j in range(16):
                    nv = jnp.full((16,), nv16[j], jnp.float32)
                    e = kb * 16 + j
                    for q in range(ch // 16):
                        sl = pl.ds(q * 16, 16)
                        rv[e, sl] = rv[e, sl] * nv
                return 0

            lax.fori_loop(0, _WIN // 16, mul16, 0)
            issue_s(wl, b)

        for h in range(nseg):
            ebase = wid * per_w + h * hlen
            wbase = wid * nwin + h * half
            pltpu.sync_copy(row_h.at[pl.ds(ebase, hlen)], row_v)
            pltpu.sync_copy(col_h.at[pl.ds(wbase, half)], col_v)
            pltpu.sync_copy(norm_h.at[pl.ds(ebase, hlen)], norm_v)
            issue_g(0, 0)

            def pair(k0, _):
                for b in range(2):
                    wl = 2 * k0 + b
                    b2 = 1 - b

                    @pl.when(wl >= 1)
                    def _():
                        wait_s(b2)

                    @pl.when(wl <= half - 2)
                    def _():
                        issue_g(wl + 1, b2)

                    wait_g(b)
                    do_mul(wl, b)
                return 0

            lax.fori_loop(0, half // 2, pair, 0)
            wait_s(1)

        plsc.subcore_barrier()
        pltpu.sync_copy(
            part.at[pl.ds(s * rows_per_sub, rows_per_sub)],
            out_h.at[c].at[pl.ds(s * rows_per_sub, rows_per_sub)])

    return pl.kernel(
        body,
        out_type=jax.ShapeDtypeStruct((2, n_pad, ch), jnp.float32),
        mesh=_mesh(),
        scratch_types=[
            pltpu.VMEM_SHARED((n_pad, ch), jnp.float32),
            pltpu.VMEM((hlen,), jnp.int32),
            pltpu.VMEM((half, _WIN), jnp.int32),
            pltpu.VMEM((hlen,), jnp.float32),
            pltpu.VMEM((_WIN, ch), jnp.float32),
            pltpu.VMEM((_WIN, ch), jnp.float32),
            pltpu.SemaphoreType.DMA,
            pltpu.SemaphoreType.DMA,
            pltpu.SemaphoreType.DMA,
            pltpu.SemaphoreType.DMA,
        ],
    )(table, row, col2, norm)


_GRID = 5


def _lstm_tail(cc, c_prev, hid):
    i = jax.nn.sigmoid(cc[:, :hid])
    f = jax.nn.sigmoid(cc[:, hid:2 * hid])
    o = jax.nn.sigmoid(cc[:, 2 * hid:3 * hid])
    g = jnp.tanh(cc[:, 3 * hid:])
    c2 = f * c_prev + i * g
    return o * jnp.tanh(c2), c2


def _enc_step_body(px0, px1, xt, ph0, ph1, hp, cp, sw, w, b, h_out, c_out):
    hid = h_out.shape[1]
    cin = xt.shape[1]
    wf = w[...]
    aggx = px0[0] + px1[0] + sw[...] * xt[...]
    aggh = ph0[0] + ph1[0] + sw[...] * hp[...]
    cc = (jnp.dot(aggx, wf[:cin], preferred_element_type=jnp.float32)
          + jnp.dot(aggh, wf[cin:], preferred_element_type=jnp.float32)
          + b[...])
    h2, c2 = _lstm_tail(cc, cp[...], hid)
    h_out[...] = h2
    c_out[...] = c2


def _enc_step0_body(px0, px1, xt, sw, w, b, h_out, c_out):
    hid = h_out.shape[1]
    cin = xt.shape[1]
    wf = w[...]
    aggx = px0[0] + px1[0] + sw[...] * xt[...]
    cc = jnp.dot(aggx, wf[:cin], preferred_element_type=jnp.float32) + b[...]
    i = jax.nn.sigmoid(cc[:, :hid])
    o = jax.nn.sigmoid(cc[:, 2 * hid:3 * hid])
    g = jnp.tanh(cc[:, 3 * hid:])
    c2 = i * g
    h_out[...] = o * jnp.tanh(c2)
    c_out[...] = c2


def _dec_step_body(vx, ph0, ph1, hp, cp, sw, w, b, h_out, c_out):
    hid = h_out.shape[1]
    wf = w[...]
    aggh = ph0[0] + ph1[0] + sw[...] * hp[...]
    cc = (jnp.dot(vx[...], wf[:hid], preferred_element_type=jnp.float32)
          + jnp.dot(aggh, wf[hid:], preferred_element_type=jnp.float32)
          + b[...])
    h2, c2 = _lstm_tail(cc, cp[...], hid)
    h_out[...] = h2
    c_out[...] = c2


def _init_body(ph0, ph1, hp, pc0, pc1, cp, sw, w, b, h_out, c_out, v_out):
    hid = h_out.shape[1]
    wf = w[...]
    aggh = ph0[0] + ph1[0] + sw[...] * hp[...]
    aggc = pc0[0] + pc1[0] + sw[...] * cp[...]
    st = (jnp.dot(aggh, wf[:hid], preferred_element_type=jnp.float32)
          + jnp.dot(aggc, wf[hid:], preferred_element_type=jnp.float32)
          + b[...])
    st = jnp.where(st > 0, st, jnp.exp(jnp.minimum(st, 0.0)) - 1.0)
    h_out[...] = st[:, :hid]
    c_out[...] = st[:, hid:]
    v_out[...] = aggh


def _row_spec(n, ch):
    return pl.BlockSpec((n // _GRID, ch), lambda i: (i, 0))


def _full_spec(r, c):
    return pl.BlockSpec((r, c), lambda i: (0, 0))


def _tc_call(body, n, hid, in_specs, num_outs):
    return pl.pallas_call(
        body,
        grid=(_GRID,),
        in_specs=in_specs,
        out_specs=[_row_spec(n, hid)] * num_outs,
        out_shape=[jax.ShapeDtypeStruct((n, hid), jnp.float32)] * num_outs,
    )


def _dis_body(deg, dis_out, sw_out):
    d = deg[0:1, :] + deg[1:2, :] + 1.0
    dis_out[...] = lax.rsqrt(d)
    sw_out[...] = 1.0 / d


def kernel(x, edge_index, edge_attr, seq_len, W_enc, b_enc, W_init, b_init,
           W_dec, b_dec):
    t_enc, n, cin = x.shape
    hid = W_enc.shape[1] // 4
    out_c = W_dec.shape[1] // 4
    e = edge_index.shape[1]

    chunk = _NW * _WIN * 2
    ep = ((e + chunk - 1) // chunk) * chunk
    pad = ep - e
    row = edge_index[0]
    col = edge_index[1]
    w = edge_attr
    if pad:
        fill = (jnp.arange(pad, dtype=jnp.int32) * 97) % n
        row = jnp.concatenate([row, fill])
        col = jnp.concatenate([col, fill])
        w = jnp.concatenate([w, jnp.zeros((pad,), jnp.float32)])

    n_pad = ((n + 127) // 128) * 128

    col2 = col.reshape(-1, _WIN)
    degp = _deg_call(col2, w.reshape(-1, _WIN), n_pad)
    dis2, sw2 = pl.pallas_call(
        _dis_body,
        in_specs=[pl.BlockSpec((2, n_pad), lambda: (0, 0))],
        out_specs=[pl.BlockSpec((1, n_pad), lambda: (0, 0))] * 2,
        out_shape=[jax.ShapeDtypeStruct((1, n_pad), jnp.float32)] * 2,
    )(degp)
    dis = dis2.reshape(n_pad)
    sw = sw2.reshape(n_pad, 1)
    norm = _norm_call(row, col, w, dis)

    agg = lambda tbl: _agg_call(tbl, row, col2, norm, n_pad)

    b_enc2 = b_enc.reshape(1, -1)
    b_init2 = b_init.reshape(1, -1)
    b_dec2 = b_dec.reshape(1, -1)

    r = n // _GRID
    rs = _row_spec(n, hid)
    ss = pl.BlockSpec((r, 1), lambda i: (i, 0))
    p0 = pl.BlockSpec((1, r, hid), lambda i: (0, i, 0))
    p1 = pl.BlockSpec((1, r, hid), lambda i: (1, i, 0))
    wspec_enc = _full_spec(cin + hid, 4 * hid)
    bspec_enc = _full_spec(1, 4 * hid)

    enc0 = _tc_call(_enc_step0_body, n, hid,
                    [p0, p1, rs, ss, wspec_enc, bspec_enc], 2)
    encs = _tc_call(_enc_step_body, n, hid,
                    [p0, p1, rs, p0, p1, rs, rs, ss, wspec_enc, bspec_enc], 2)
    init = pl.pallas_call(
        _init_body,
        grid=(_GRID,),
        in_specs=[p0, p1, rs, p0, p1, rs, ss,
                  _full_spec(2 * hid, 2 * out_c), _full_spec(1, 2 * out_c)],
        out_specs=[_row_spec(n, out_c)] * 3,
        out_shape=[jax.ShapeDtypeStruct((n, out_c), jnp.float32)] * 3,
    )
    decs = _tc_call(_dec_step_body, n, out_c,
                    [rs, p0, p1, rs, rs, ss,
                     _full_spec(hid + out_c, 4 * out_c),
                     _full_spec(1, 4 * out_c)], 2)

    px = agg(x[0])
    h, c = enc0(px, px, x[0], sw, W_enc, b_enc2)
    for t in range(1, t_enc):
        px = agg(x[t])
        ph = agg(h)
        h, c = encs(px, px, x[t], ph, ph, h, c, sw, W_enc, b_enc2)

    ph = agg(h)
    pc = agg(c)
    h0, c0, v = init(ph, ph, h, pc, pc, c, sw, W_init, b_init2)

    outs = jnp.zeros((t_enc, n, out_c), x.dtype)

    def dec_body(t, carry):
        h_t, c_t, acc = carry
        p = agg(h_t)
        h2, c2 = decs(v, p, p, h_t, c_t, sw, W_dec, b_dec2)
        return (h2, c2, acc.at[t].set(h2))

    _, _, outs = lax.fori_loop(0, seq_len, dec_body, (h0, c0, outs))
    return outs


# speedup vs baseline: 1.0598x; 1.0005x over previous
import functools

import jax
import jax.numpy as jnp
from jax import lax
from jax.experimental import pallas as pl
from jax.experimental.pallas import tpu as pltpu
from jax.experimental.pallas import tpu_sc as plsc

_WIN = 128
_NW = 32
_NSUB = 16


def _mesh():
    return plsc.VectorSubcoreMesh(core_axis_name="c", subcore_axis_name="s")


def _deg_call(col2, w2, n_pad):
    ep = w2.shape[0] * w2.shape[1]
    per_w = ep // _NW
    nwin = per_w // _WIN

    def body(col_h, w_h, out_h, acc, col_v, w_v, zero_v, sem):
        c = lax.axis_index("c")
        s = lax.axis_index("s")
        wid = c * _NSUB + s

        def _zero(j, _):
            zero_v[pl.ds(j * 16, 16)] = jnp.zeros((16,), jnp.float32)
            return 0

        lax.fori_loop(0, n_pad // 16, _zero, 0)

        @pl.when(s == 0)
        def _():
            pltpu.sync_copy(zero_v, acc)

        plsc.subcore_barrier()

        wbase = wid * nwin
        pltpu.sync_copy(col_h.at[pl.ds(wbase, nwin)], col_v)
        pltpu.sync_copy(w_h.at[pl.ds(wbase, nwin)], w_v)

        def win(k, _):
            pltpu.async_copy(
                w_v.at[k], acc.at[col_v.at[k]], sem, add=True)
            return 0

        lax.fori_loop(0, nwin, win, 0)

        def drain(k, _):
            pltpu.make_async_copy(
                w_v.at[0], acc.at[col_v.at[0]], sem).wait()
            return 0

        lax.fori_loop(0, nwin, drain, 0)
        plsc.subcore_barrier()

        @pl.when(s == 0)
        def _():
            pltpu.sync_copy(acc, out_h.at[c])

    return pl.kernel(
        body,
        out_type=jax.ShapeDtypeStruct((2, n_pad), jnp.float32),
        mesh=_mesh(),
        scratch_types=[
            pltpu.VMEM_SHARED((n_pad,), jnp.float32),
            pltpu.VMEM((nwin, _WIN), jnp.int32),
            pltpu.VMEM((nwin, _WIN), jnp.float32),
            pltpu.VMEM((n_pad,), jnp.float32),
            pltpu.SemaphoreType.DMA,
        ],
    )(col2, w2)


def _norm_call(row, col, w, dis):
    ep = row.shape[0]
    per_w = ep // _NW
    nwin = per_w // _WIN

    def body(row_h, col_h, w_h, dis_h, out_h, row_v, col_v, w_v,
             dr0, dr1, dc0, dc1, nv0, nv1, sg0, sg1, ss0, ss1):
        c = lax.axis_index("c")
        s = lax.axis_index("s")
        wid = c * _NSUB + s
        ebase = wid * per_w
        dr = (dr0, dr1)
        dc = (dc0, dc1)
        nv = (nv0, nv1)
        sg = (sg0, sg1)
        ss = (ss0, ss1)

        pltpu.sync_copy(row_h.at[pl.ds(ebase, per_w)], row_v)
        pltpu.sync_copy(col_h.at[pl.ds(ebase, per_w)], col_v)
        pltpu.sync_copy(w_h.at[pl.ds(ebase, per_w)], w_v)

        def issue_g(wl, b):
            pltpu.async_copy(
                dis_h.at[row_v.at[pl.ds(wl * _WIN, _WIN)]], dr[b], sg[b])
            pltpu.async_copy(
                dis_h.at[col_v.at[pl.ds(wl * _WIN, _WIN)]], dc[b], sg[b])

        def wait_g(b):
            pltpu.make_async_copy(
                dis_h.at[row_v.at[pl.ds(0, _WIN)]], dr[b], sg[b]).wait()
            pltpu.make_async_copy(
                dis_h.at[col_v.at[pl.ds(0, _WIN)]], dc[b], sg[b]).wait()

        def issue_s(wl, b):
            pltpu.async_copy(
                nv[b], out_h.at[pl.ds(ebase + wl * _WIN, _WIN)], ss[b])

        def wait_s(b):
            pltpu.make_async_copy(
                nv[b], out_h.at[pl.ds(ebase, _WIN)], ss[b]).wait()

        issue_g(0, 0)

        def pair(k0, _):
            for b in range(2):
                wl = 2 * k0 + b
                b2 = 1 - b

                @pl.when(wl >= 1)
                def _():
                    wait_s(b2)

                @pl.when(wl <= nwin - 2)
                def _():
                    issue_g(wl + 1, b2)

                wait_g(b)
                for q in range(_WIN // 16):
                    sl = pl.ds(q * 16, 16)
                    wsl = pl.ds(wl * _WIN + q * 16, 16)
                    nv[b][sl] = dr[b][sl] * w_v[wsl] * dc[b][sl]
                issue_s(wl, b)
            return 0

        lax.fori_loop(0, nwin // 2, pair, 0)
        wait_s(1)

    return pl.kernel(
        body,
        out_type=jax.ShapeDtypeStruct((ep,), jnp.float32),
        mesh=_mesh(),
        scratch_types=[
            pltpu.VMEM((per_w,), jnp.int32),
            pltpu.VMEM((per_w,), jnp.int32),
            pltpu.VMEM((per_w,), jnp.float32),
            pltpu.VMEM((_WIN,), jnp.float32),
            pltpu.VMEM((_WIN,), jnp.float32),
            pltpu.VMEM((_WIN,), jnp.float32),
            pltpu.VMEM((_WIN,), jnp.float32),
            pltpu.VMEM((_WIN,), jnp.float32),
            pltpu.VMEM((_WIN,), jnp.float32),
            pltpu.SemaphoreType.DMA,
            pltpu.SemaphoreType.DMA,
            pltpu.SemaphoreType.DMA,
            pltpu.SemaphoreType.DMA,
        ],
    )(row, col, w, dis)


def _agg_call(table, row, col2, norm, n_pad):
    n_nodes, ch = table.shape
    ep = row.shape[0]
    per_w = ep // _NW
    nwin = per_w // _WIN
    nseg = 2
    half = nwin // nseg
    hlen = half * _WIN
    rows_per_sub = n_pad // _NSUB

    zfull = rows_per_sub // _WIN
    zrem = rows_per_sub % _WIN

    def body(table_h, row_h, col_h, norm_h, out_h, part, row_v, col_v,
             norm_v, rows_v0, rows_v1, sg0, sg1, ss0, ss1):
        c = lax.axis_index("c")
        s = lax.axis_index("s")
        wid = c * _NSUB + s
        rows_b = (rows_v0, rows_v1)
        sg = (sg0, sg1)
        ss = (ss0, ss1)

        def _zero(j, _):
            for q in range(ch // 16):
                rows_v0[j, pl.ds(q * 16, 16)] = jnp.zeros((16,), jnp.float32)
            return 0

        lax.fori_loop(0, _WIN, _zero, 0)
        for k in range(zfull):
            pltpu.sync_copy(
                rows_v0, part.at[pl.ds(s * rows_per_sub + k * _WIN, _WIN)])
        if zrem:
            pltpu.sync_copy(
                rows_v0.at[pl.ds(0, zrem)],
                part.at[pl.ds(s * rows_per_sub + zfull * _WIN, zrem)])
        plsc.subcore_barrier()

        def issue_g(wl, b):
            return pltpu.async_copy(
                table_h.at[row_v.at[pl.ds(wl * _WIN, _WIN)]], rows_b[b], sg[b])

        def wait_g(b):
            pltpu.make_async_copy(
                table_h.at[row_v.at[pl.ds(0, _WIN)]], rows_b[b], sg[b]).wait()

        def issue_s(wl, b):
            return pltpu.async_copy(
                rows_b[b], part.at[col_v.at[wl]], ss[b], add=True)

        def wait_s(b):
            pltpu.make_async_copy(
                rows_b[b], part.at[col_v.at[0]], ss[b]).wait()

        def do_mul(wl, b):
            rv = rows_b[b]

            def mul16(kb, _):
                nv16 = norm_v[pl.ds(wl * _WIN + kb * 16, 16)]
                for j in range(16):
                    nv = jnp.full((16,), nv16[j], jnp.float32)
                    e = kb * 16 + j
                    for q in range(ch // 16):
                        sl = pl.ds(q * 16, 16)
                        rv[e, sl] = rv[e, sl] * nv
                return 0

            lax.fori_loop(0, _WIN // 16, mul16, 0)
            issue_s(wl, b)

        for h in range(nseg):
            ebase = wid * per_w + h * hlen
            wbase = wid * nwin + h * half
            pltpu.sync_copy(row_h.at[pl.ds(ebase, hlen)], row_v)
            pltpu.sync_copy(col_h.at[pl.ds(wbase, half)], col_v)
            pltpu.sync_copy(norm_h.at[pl.ds(ebase, hlen)], norm_v)
            issue_g(0, 0)

            def pair(k0, _):
                for b in range(2):
                    wl = 2 * k0 + b
                    b2 = 1 - b

                    @pl.when(wl >= 1)
                    def _():
                        wait_s(b2)

                    @pl.when(wl <= half - 2)
                    def _():
                        issue_g(wl + 1, b2)

                    wait_g(b)
                    do_mul(wl, b)
                return 0

            lax.fori_loop(0, half // 2, pair, 0)
            wait_s(1)

        plsc.subcore_barrier()
        pltpu.sync_copy(
            part.at[pl.ds(s * rows_per_sub, rows_per_sub)],
            out_h.at[c].at[pl.ds(s * rows_per_sub, rows_per_sub)])

    return pl.kernel(
        body,
        out_type=jax.ShapeDtypeStruct((2, n_pad, ch), jnp.float32),
        mesh=_mesh(),
        scratch_types=[
            pltpu.VMEM_SHARED((n_pad, ch), jnp.float32),
            pltpu.VMEM((hlen,), jnp.int32),
            pltpu.VMEM((half, _WIN), jnp.int32),
            pltpu.VMEM((hlen,), jnp.float32),
            pltpu.VMEM((_WIN, ch), jnp.float32),
            pltpu.VMEM((_WIN, ch), jnp.float32),
            pltpu.SemaphoreType.DMA,
            pltpu.SemaphoreType.DMA,
            pltpu.SemaphoreType.DMA,
            pltpu.SemaphoreType.DMA,
        ],
    )(table, row, col2, norm)


_GRID = 2


def _lstm_tail(cc, c_prev, hid):
    i = jax.nn.sigmoid(cc[:, :hid])
    f = jax.nn.sigmoid(cc[:, hid:2 * hid])
    o = jax.nn.sigmoid(cc[:, 2 * hid:3 * hid])
    g = jnp.tanh(cc[:, 3 * hid:])
    c2 = f * c_prev + i * g
    return o * jnp.tanh(c2), c2


def _enc_step_body(px0, px1, xt, ph0, ph1, hp, cp, sw, w, b, h_out, c_out):
    hid = h_out.shape[1]
    cin = xt.shape[1]
    wf = w[...]
    aggx = px0[0] + px1[0] + sw[...] * xt[...]
    aggh = ph0[0] + ph1[0] + sw[...] * hp[...]
    cc = (jnp.dot(aggx, wf[:cin], preferred_element_type=jnp.float32)
          + jnp.dot(aggh, wf[cin:], preferred_element_type=jnp.float32)
          + b[...])
    h2, c2 = _lstm_tail(cc, cp[...], hid)
    h_out[...] = h2
    c_out[...] = c2


def _enc_step0_body(px0, px1, xt, sw, w, b, h_out, c_out):
    hid = h_out.shape[1]
    cin = xt.shape[1]
    wf = w[...]
    aggx = px0[0] + px1[0] + sw[...] * xt[...]
    cc = jnp.dot(aggx, wf[:cin], preferred_element_type=jnp.float32) + b[...]
    i = jax.nn.sigmoid(cc[:, :hid])
    o = jax.nn.sigmoid(cc[:, 2 * hid:3 * hid])
    g = jnp.tanh(cc[:, 3 * hid:])
    c2 = i * g
    h_out[...] = o * jnp.tanh(c2)
    c_out[...] = c2


def _dec_step_body(vx, ph0, ph1, hp, cp, sw, w, b, h_out, c_out):
    hid = h_out.shape[1]
    wf = w[...]
    aggh = ph0[0] + ph1[0] + sw[...] * hp[...]
    cc = (jnp.dot(vx[...], wf[:hid], preferred_element_type=jnp.float32)
          + jnp.dot(aggh, wf[hid:], preferred_element_type=jnp.float32)
          + b[...])
    h2, c2 = _lstm_tail(cc, cp[...], hid)
    h_out[...] = h2
    c_out[...] = c2


def _init_body(ph0, ph1, hp, pc0, pc1, cp, sw, w, b, h_out, c_out, v_out):
    hid = h_out.shape[1]
    wf = w[...]
    aggh = ph0[0] + ph1[0] + sw[...] * hp[...]
    aggc = pc0[0] + pc1[0] + sw[...] * cp[...]
    st = (jnp.dot(aggh, wf[:hid], preferred_element_type=jnp.float32)
          + jnp.dot(aggc, wf[hid:], preferred_element_type=jnp.float32)
          + b[...])
    st = jnp.where(st > 0, st, jnp.exp(jnp.minimum(st, 0.0)) - 1.0)
    h_out[...] = st[:, :hid]
    c_out[...] = st[:, hid:]
    v_out[...] = aggh


def _row_spec(n, ch):
    return pl.BlockSpec((n // _GRID, ch), lambda i: (i, 0))


def _full_spec(r, c):
    return pl.BlockSpec((r, c), lambda i: (0, 0))


def _tc_call(body, n, hid, in_specs, num_outs):
    return pl.pallas_call(
        body,
        grid=(_GRID,),
        in_specs=in_specs,
        out_specs=[_row_spec(n, hid)] * num_outs,
        out_shape=[jax.ShapeDtypeStruct((n, hid), jnp.float32)] * num_outs,
    )


def _dis_body(deg, dis_out, sw_out):
    d = deg[0:1, :] + deg[1:2, :] + 1.0
    dis_out[...] = lax.rsqrt(d)
    sw_out[...] = 1.0 / d


def kernel(x, edge_index, edge_attr, seq_len, W_enc, b_enc, W_init, b_init,
           W_dec, b_dec):
    t_enc, n, cin = x.shape
    hid = W_enc.shape[1] // 4
    out_c = W_dec.shape[1] // 4
    e = edge_index.shape[1]

    chunk = _NW * _WIN * 2
    ep = ((e + chunk - 1) // chunk) * chunk
    pad = ep - e
    row = edge_index[0]
    col = edge_index[1]
    w = edge_attr
    if pad:
        fill = (jnp.arange(pad, dtype=jnp.int32) * 97) % n
        row = jnp.concatenate([row, fill])
        col = jnp.concatenate([col, fill])
        w = jnp.concatenate([w, jnp.zeros((pad,), jnp.float32)])

    n_pad = ((n + 127) // 128) * 128

    col2 = col.reshape(-1, _WIN)
    degp = _deg_call(col2, w.reshape(-1, _WIN), n_pad)
    dis2, sw2 = pl.pallas_call(
        _dis_body,
        in_specs=[pl.BlockSpec((2, n_pad), lambda: (0, 0))],
        out_specs=[pl.BlockSpec((1, n_pad), lambda: (0, 0))] * 2,
        out_shape=[jax.ShapeDtypeStruct((1, n_pad), jnp.float32)] * 2,
    )(degp)
    dis = dis2.reshape(n_pad)
    sw = sw2.reshape(n_pad, 1)
    norm = _norm_call(row, col, w, dis)

    agg = lambda tbl: _agg_call(tbl, row, col2, norm, n_pad)

    b_enc2 = b_enc.reshape(1, -1)
    b_init2 = b_init.reshape(1, -1)
    b_dec2 = b_dec.reshape(1, -1)

    r = n // _GRID
    rs = _row_spec(n, hid)
    ss = pl.BlockSpec((r, 1), lambda i: (i, 0))
    p0 = pl.BlockSpec((1, r, hid), lambda i: (0, i, 0))
    p1 = pl.BlockSpec((1, r, hid), lambda i: (1, i, 0))
    wspec_enc = _full_spec(cin + hid, 4 * hid)
    bspec_enc = _full_spec(1, 4 * hid)

    enc0 = _tc_call(_enc_step0_body, n, hid,
                    [p0, p1, rs, ss, wspec_enc, bspec_enc], 2)
    encs = _tc_call(_enc_step_body, n, hid,
                    [p0, p1, rs, p0, p1, rs, rs, ss, wspec_enc, bspec_enc], 2)
    init = pl.pallas_call(
        _init_body,
        grid=(_GRID,),
        in_specs=[p0, p1, rs, p0, p1, rs, ss,
                  _full_spec(2 * hid, 2 * out_c), _full_spec(1, 2 * out_c)],
        out_specs=[_row_spec(n, out_c)] * 3,
        out_shape=[jax.ShapeDtypeStruct((n, out_c), jnp.float32)] * 3,
    )
    decs = _tc_call(_dec_step_body, n, out_c,
                    [rs, p0, p1, rs, rs, ss,
                     _full_spec(hid + out_c, 4 * out_c),
                     _full_spec(1, 4 * out_c)], 2)

    px = agg(x[0])
    h, c = enc0(px, px, x[0], sw, W_enc, b_enc2)
    for t in range(1, t_enc):
        px = agg(x[t])
        ph = agg(h)
        h, c = encs(px, px, x[t], ph, ph, h, c, sw, W_enc, b_enc2)

    ph = agg(h)
    pc = agg(c)
    h0, c0, v = init(ph, ph, h, pc, pc, c, sw, W_init, b_init2)

    outs = jnp.zeros((t_enc, n, out_c), x.dtype)

    def dec_body(t, carry):
        h_t, c_t, acc = carry
        p = agg(h_t)
        h2, c2 = decs(v, p, p, h_t, c_t, sw, W_dec, b_dec2)
        return (h2, c2, acc.at[t].set(h2))

    _, _, outs = lax.fori_loop(0, seq_len, dec_body, (h0, c0, outs))
    return outs
